# baseline (speedup 1.0000x reference)
import functools
import math

import jax
import jax.numpy as jnp
from jax import lax
from jax.experimental import pallas as pl
from jax.experimental.pallas import tpu as pltpu
from jax.experimental.pallas import tpu_sc as plsc

NC, NS, L = 2, 16, 16
NW = NC * NS


def _make_sc_kernel(B, C, HW, K):
    QW = NW // B
    PPW = K // QW
    PPWP = ((PPW + L - 1) // L) * L
    CCH = C // L
    JCH = PPWP // L

    mesh = plsc.VectorSubcoreMesh(core_axis_name="c", subcore_axis_name="s")

    @functools.partial(
        pl.kernel,
        out_type=jax.ShapeDtypeStruct((NW, 2, L), jnp.float32),
        mesh=mesh,
        compiler_params=pltpu.CompilerParams(needs_layout_passes=False),
        scratch_types=[
            pltpu.VMEM((K,), jnp.int32),
            pltpu.VMEM((K,), jnp.int32),
            pltpu.VMEM((PPWP,), jnp.float32),
            pltpu.VMEM((PPWP,), jnp.int32),
            pltpu.VMEM((PPWP,), jnp.int32),
            pltpu.VMEM((PPWP, C), jnp.float32),
            pltpu.VMEM((PPWP, C), jnp.float32),
            pltpu.VMEM((2, L), jnp.float32),
            pltpu.SemaphoreType.DMA,
            pltpu.SemaphoreType.DMA,
        ],
    )
    def sc_kernel(feat_hbm, ind_hbm, mask_hbm, tgtT_hbm, out_hbm,
                  ind_v, maski_v, maskf_v, rows_v, trows_v, gath_v, tgt_v,
                  part_v, sem0, sem1):
        tgt2d_hbm = tgtT_hbm.reshape(K * B, C)
        wid = lax.axis_index("s") * NC + lax.axis_index("c")
        lane = jnp.arange(L, dtype=jnp.int32)
        b_s = wid // QW
        k0 = (wid % QW) * PPW

        pltpu.sync_copy(ind_hbm.at[b_s], ind_v)
        pltpu.sync_copy(mask_hbm.at[b_s], maski_v)

        bbase = jnp.full((L,), b_s * HW, jnp.int32)
        cnt = jnp.zeros((L,), jnp.float32)
        for jj in range(JCH):
            kk = k0 + jj * L + lane
            kks = jnp.minimum(kk, K - 1)
            rows_v[pl.ds(jj * L, L)] = bbase + plsc.load_gather(ind_v, [kks])
            trows_v[pl.ds(jj * L, L)] = kks * B + b_s
            mi = plsc.load_gather(maski_v, [kks])
            ok = jnp.logical_and(kk < k0 + PPW, mi > 0)
            mfv = jnp.where(ok, 1.0, 0.0).astype(jnp.float32)
            maskf_v[pl.ds(jj * L, L)] = mfv
            cnt = cnt + mfv

        tgt_cp = pltpu.async_copy(tgt2d_hbm.at[trows_v], tgt_v, sem1)
        pltpu.async_copy(feat_hbm.at[rows_v], gath_v, sem0).wait()
        tgt_cp.wait()

        def _gat1(ref, j):
            return plsc.load_gather(ref, [jnp.full((L,), j, jnp.int32)])

        @plsc.parallel_loop(0, PPW, unroll=2,
                            carry=jnp.zeros((L,), jnp.float32))
        def acc(j, a):
            mf = _gat1(maskf_v, j)
            s = jnp.zeros((L,), jnp.float32)
            for cc in range(CCH):
                d = (gath_v[j, pl.ds(cc * L, L)]
                     - tgt_v[j, pl.ds(cc * L, L)])
                s = s + d * d
            return a + s * mf

        part_v[0, :] = acc
        part_v[1, :] = cnt
        pltpu.sync_copy(part_v, out_hbm.at[wid])

    return sc_kernel


def kernel(output, mask, ind, target):
    B, C, H, W = output.shape
    K = ind.shape[1]
    HW = H * W

    feat = jnp.transpose(output, (0, 2, 3, 1)).reshape(B * HW, C)
    tgtT = jnp.transpose(target, (1, 0, 2))

    sck = _make_sc_kernel(B, C, HW, K)
    parts = sck(feat, ind.astype(jnp.int32), mask.astype(jnp.int32), tgtT)

    sumsq = jnp.sum(parts[:, 0, :])
    cnt = jnp.sum(parts[:, 1, :])
    denom = jnp.maximum(cnt * C, 1.0)
    return jnp.where(cnt > 0, sumsq / denom, jnp.asarray(0.0, jnp.float32))


# speedup vs baseline: 1.0294x; 1.0294x over previous
import functools
import math

import jax
import jax.numpy as jnp
from jax import lax
from jax.experimental import pallas as pl
from jax.experimental.pallas import tpu as pltpu
from jax.experimental.pallas import tpu_sc as plsc

NC, NS, L = 2, 16, 16
NW = NC * NS


def _make_sc_kernel(B, C, HW, K):
    QW = NW // B
    PPW = K // QW
    PPWP = ((PPW + L - 1) // L) * L
    CCH = C // L
    JCH = PPWP // L

    mesh = plsc.VectorSubcoreMesh(core_axis_name="c", subcore_axis_name="s")

    @functools.partial(
        pl.kernel,
        out_type=jax.ShapeDtypeStruct((NW, 2, L), jnp.float32),
        mesh=mesh,
        compiler_params=pltpu.CompilerParams(needs_layout_passes=False),
        scratch_types=[
            pltpu.VMEM((K,), jnp.int32),
            pltpu.VMEM((K,), jnp.int32),
            pltpu.VMEM((PPWP,), jnp.float32),
            pltpu.VMEM((PPWP,), jnp.int32),
            pltpu.VMEM((PPWP,), jnp.int32),
            pltpu.VMEM((PPWP, C), jnp.float32),
            pltpu.VMEM((PPWP, C), jnp.float32),
            pltpu.VMEM((2, L), jnp.float32),
            pltpu.SemaphoreType.DMA,
            pltpu.SemaphoreType.DMA,
        ],
    )
    def sc_kernel(feat_hbm, ind_hbm, mask_hbm, tgtT_hbm, out_hbm,
                  ind_v, maski_v, maskf_v, rows_v, trows_v, gath_v, tgt_v,
                  part_v, sem0, sem1):
        tgt2d_hbm = tgtT_hbm.reshape(K * B, C)
        wid = lax.axis_index("s") * NC + lax.axis_index("c")
        lane = jnp.arange(L, dtype=jnp.int32)
        b_s = wid // QW
        k0 = (wid % QW) * PPW

        ind_cp = pltpu.async_copy(ind_hbm.at[b_s], ind_v, sem0)
        pltpu.async_copy(mask_hbm.at[b_s], maski_v, sem1).wait()
        ind_cp.wait()

        bbase = jnp.full((L,), b_s * HW, jnp.int32)
        cnt = jnp.zeros((L,), jnp.float32)
        for jj in range(JCH):
            kk = k0 + jj * L + lane
            kks = jnp.minimum(kk, K - 1)
            rows_v[pl.ds(jj * L, L)] = bbase + plsc.load_gather(ind_v, [kks])
            trows_v[pl.ds(jj * L, L)] = kks * B + b_s
            mi = plsc.load_gather(maski_v, [kks])
            ok = jnp.logical_and(kk < k0 + PPW, mi > 0)
            mfv = jnp.where(ok, 1.0, 0.0).astype(jnp.float32)
            maskf_v[pl.ds(jj * L, L)] = mfv
            cnt = cnt + mfv

        tgt_cp = pltpu.async_copy(tgt2d_hbm.at[trows_v], tgt_v, sem1)
        pltpu.async_copy(feat_hbm.at[rows_v], gath_v, sem0).wait()
        tgt_cp.wait()

        def _gat1(ref, j):
            return plsc.load_gather(ref, [jnp.full((L,), j, jnp.int32)])

        def mse_body(j, a):
            mf = _gat1(maskf_v, j)
            s = jnp.zeros((L,), jnp.float32)
            for cc in range(CCH):
                d = (gath_v[j, pl.ds(cc * L, L)]
                     - tgt_v[j, pl.ds(cc * L, L)])
                s = s + d * d
            return a + s * mf

        acc = lax.fori_loop(0, PPW, mse_body, jnp.zeros((L,), jnp.float32))

        part_v[0, :] = acc
        part_v[1, :] = cnt
        pltpu.sync_copy(part_v, out_hbm.at[wid])

    return sc_kernel


def kernel(output, mask, ind, target):
    B, C, H, W = output.shape
    K = ind.shape[1]
    HW = H * W

    feat = jnp.transpose(output, (0, 2, 3, 1)).reshape(B * HW, C)
    tgtT = jnp.transpose(target, (1, 0, 2))

    sck = _make_sc_kernel(B, C, HW, K)
    parts = sck(feat, ind.astype(jnp.int32), mask.astype(jnp.int32), tgtT)

    sumsq = jnp.sum(parts[:, 0, :])
    cnt = jnp.sum(parts[:, 1, :])
    denom = jnp.maximum(cnt * C, 1.0)
    return jnp.where(cnt > 0, sumsq / denom, jnp.asarray(0.0, jnp.float32))
